# Initial kernel scaffold; baseline (speedup 1.0000x reference)
import jax
import jax.numpy as jnp
from jax.experimental import pallas as pl


def kernel(quantized_latent_embedding, edge_index):
    raise NotImplementedError("write your pallas kernel here")


# speedup vs baseline: 1.3376x; 1.3376x over previous
import functools

import jax
import jax.numpy as jnp
from jax import lax
from jax.experimental import pallas as pl
from jax.experimental.pallas import tpu as pltpu
from jax.experimental.pallas import tpu_sc as plsc

V = 10000
D = 128
B = 320000
NC, NS = 2, 16
NW = NC * NS
E_PER_W = B // NW
C = 80
NCH = E_PER_W // C
GRP = C // 16


def _body(table, src_i, dst_i, out, idx_s, idx_d, rows_s, rows_d, out_v,
          sem_s0, sem_s1, sem_d0, sem_d1):
    wid = lax.axis_index("c") * NS + lax.axis_index("s")

    pltpu.sync_copy(src_i.at[wid], idx_s)
    pltpu.sync_copy(dst_i.at[wid], idx_d)

    sems = ((sem_s0, sem_d0), (sem_s1, sem_d1))

    def start(g, b):
        ss, sd = sems[b]
        pltpu.async_copy(table.at[idx_s.at[g]], rows_s.at[b], ss)
        pltpu.async_copy(table.at[idx_d.at[g]], rows_d.at[b], sd)

    def compute(g, b):
        ss, sd = sems[b]
        pltpu.make_async_copy(table.at[idx_s.at[g]], rows_s.at[b], ss).wait()
        pltpu.make_async_copy(table.at[idx_d.at[g]], rows_d.at[b], sd).wait()
        rs = rows_s.at[b]
        rd = rows_d.at[b]
        for grp in range(GRP):
            eidx = jnp.arange(16, dtype=jnp.int32) + (grp * 16)

            def dbody(d, acc):
                dd = jnp.full((16,), d, dtype=jnp.int32)
                sv = plsc.load_gather(rs, [eidx, dd])
                dv = plsc.load_gather(rd, [eidx, dd])
                return acc + sv * dv

            acc = lax.fori_loop(0, D, dbody, jnp.zeros((16,), jnp.float32),
                                unroll=8)
            res = 1.0 / (1.0 + jnp.exp(-acc))
            out_v[pl.ds(g * C + grp * 16, 16)] = res

    start(0, 0)
    start(1, 1)
    compute(0, 0)
    start(2, 0)
    compute(1, 1)
    start(3, 1)

    def pair(p, carry):
        g0 = 2 * p
        compute(g0, 0)
        start(g0 + 2, 0)
        compute(g0 + 1, 1)

        @pl.when(g0 + 3 < NCH)
        def _():
            start(g0 + 3, 1)

        return carry

    lax.fori_loop(1, NCH // 2, pair, 0)
    compute(NCH - 1, 0)

    pltpu.sync_copy(out_v, out.at[pl.ds(wid * E_PER_W, E_PER_W)])


@functools.partial(jax.jit, donate_argnums=())
def _decode(table, src_i, dst_i):
    run = functools.partial(
        pl.kernel,
        out_type=jax.ShapeDtypeStruct((B,), jnp.float32),
        mesh=plsc.VectorSubcoreMesh(core_axis_name="c", subcore_axis_name="s"),
        compiler_params=pltpu.CompilerParams(
            needs_layout_passes=False, use_tc_tiling_on_sc=False),
        scratch_types=[
            pltpu.VMEM((NCH, C), jnp.int32),
            pltpu.VMEM((NCH, C), jnp.int32),
            pltpu.VMEM((2, C, D), jnp.float32),
            pltpu.VMEM((2, C, D), jnp.float32),
            pltpu.VMEM((E_PER_W,), jnp.float32),
            pltpu.SemaphoreType.DMA,
            pltpu.SemaphoreType.DMA,
            pltpu.SemaphoreType.DMA,
            pltpu.SemaphoreType.DMA,
        ],
    )(_body)
    return run(table, src_i, dst_i)


def kernel(quantized_latent_embedding, edge_index):
    src_i = edge_index[0].astype(jnp.int32).reshape(NW, NCH, C)
    dst_i = edge_index[1].astype(jnp.int32).reshape(NW, NCH, C)
    return _decode(quantized_latent_embedding, src_i, dst_i)


# speedup vs baseline: 9.1362x; 6.8301x over previous
import functools

import jax
import jax.numpy as jnp
from jax import lax
from jax.experimental import pallas as pl
from jax.experimental.pallas import tpu as pltpu
from jax.experimental.pallas import tpu_sc as plsc

V = 10000
D = 128
B = 320000
NC, NS = 2, 16
NW = NC * NS
E_PER_W = B // NW
C = 80
NCH = E_PER_W // C
GRP = C // 16


def _body(table, src_i, dst_i, out, idx_s, idx_d, rows_s, rows_d, out_v,
          sem_s0, sem_s1, sem_d0, sem_d1):
    wid = lax.axis_index("c") * NS + lax.axis_index("s")

    pltpu.sync_copy(src_i.at[wid], idx_s)
    pltpu.sync_copy(dst_i.at[wid], idx_d)

    sems = ((sem_s0, sem_d0), (sem_s1, sem_d1))

    def start(g, b):
        ss, sd = sems[b]
        pltpu.async_copy(table.at[idx_s.at[g]], rows_s.at[b], ss)
        pltpu.async_copy(table.at[idx_d.at[g]], rows_d.at[b], sd)

    def compute(g, b):
        ss, sd = sems[b]
        pltpu.make_async_copy(table.at[idx_s.at[g]], rows_s.at[b], ss).wait()
        pltpu.make_async_copy(table.at[idx_d.at[g]], rows_d.at[b], sd).wait()
        rs = rows_s.at[b]
        rd = rows_d.at[b]
        lanes = jnp.arange(16, dtype=jnp.int32)
        for grp in range(GRP):
            eidx = lanes + (grp * 16)

            def dbody(d, acc):
                dd = (lanes + d) & 127
                sv = plsc.load_gather(rs, [eidx, dd])
                dv = plsc.load_gather(rd, [eidx, dd])
                return acc + sv * dv

            acc = lax.fori_loop(0, D, dbody, jnp.zeros((16,), jnp.float32),
                                unroll=8)
            res = 1.0 / (1.0 + jnp.exp(-acc))
            out_v[pl.ds(g * C + grp * 16, 16)] = res

    start(0, 0)
    start(1, 1)
    compute(0, 0)
    start(2, 0)
    compute(1, 1)
    start(3, 1)

    def pair(p, carry):
        g0 = 2 * p
        compute(g0, 0)
        start(g0 + 2, 0)
        compute(g0 + 1, 1)

        @pl.when(g0 + 3 < NCH)
        def _():
            start(g0 + 3, 1)

        return carry

    lax.fori_loop(1, NCH // 2, pair, 0)
    compute(NCH - 1, 0)

    pltpu.sync_copy(out_v, out.at[pl.ds(wid * E_PER_W, E_PER_W)])


@functools.partial(jax.jit, donate_argnums=())
def _decode(table, src_i, dst_i):
    run = functools.partial(
        pl.kernel,
        out_type=jax.ShapeDtypeStruct((B,), jnp.float32),
        mesh=plsc.VectorSubcoreMesh(core_axis_name="c", subcore_axis_name="s"),
        compiler_params=pltpu.CompilerParams(
            needs_layout_passes=False, use_tc_tiling_on_sc=False),
        scratch_types=[
            pltpu.VMEM((NCH, C), jnp.int32),
            pltpu.VMEM((NCH, C), jnp.int32),
            pltpu.VMEM((2, C, D), jnp.float32),
            pltpu.VMEM((2, C, D), jnp.float32),
            pltpu.VMEM((E_PER_W,), jnp.float32),
            pltpu.SemaphoreType.DMA,
            pltpu.SemaphoreType.DMA,
            pltpu.SemaphoreType.DMA,
            pltpu.SemaphoreType.DMA,
        ],
    )(_body)
    return run(table, src_i, dst_i)


def kernel(quantized_latent_embedding, edge_index):
    src_i = edge_index[0].astype(jnp.int32).reshape(NW, NCH, C)
    dst_i = edge_index[1].astype(jnp.int32).reshape(NW, NCH, C)
    return _decode(quantized_latent_embedding, src_i, dst_i)


# speedup vs baseline: 10.8735x; 1.1902x over previous
import functools

import jax
import jax.numpy as jnp
from jax import lax
from jax.experimental import pallas as pl
from jax.experimental.pallas import tpu as pltpu
from jax.experimental.pallas import tpu_sc as plsc

V = 10000
D = 128
B = 320000
NC, NS = 2, 16
NW = NC * NS
E_PER_W = B // NW
C = 80
NCH = E_PER_W // C
GRP = C // 16


NBUF = 4


def _body(table, src_i, dst_i, out, idx_s, idx_d, rows_s, rows_d, out_v,
          *sems_flat):
    wid = lax.axis_index("c") * NS + lax.axis_index("s")

    pltpu.sync_copy(src_i.at[wid], idx_s)
    pltpu.sync_copy(dst_i.at[wid], idx_d)

    sems = tuple(zip(sems_flat[:NBUF], sems_flat[NBUF:]))

    def start(g, b):
        ss, sd = sems[b]
        pltpu.async_copy(table.at[idx_s.at[g]], rows_s.at[b], ss)
        pltpu.async_copy(table.at[idx_d.at[g]], rows_d.at[b], sd)

    def compute(g, b):
        ss, sd = sems[b]
        pltpu.make_async_copy(table.at[idx_s.at[g]], rows_s.at[b], ss).wait()
        pltpu.make_async_copy(table.at[idx_d.at[g]], rows_d.at[b], sd).wait()
        rs = rows_s.at[b]
        rd = rows_d.at[b]
        lanes = jnp.arange(16, dtype=jnp.int32)
        for grp in range(GRP):
            eidx = lanes + (grp * 16)

            def dbody(d, acc):
                dd = (lanes + d) & 127
                sv = plsc.load_gather(rs, [eidx, dd])
                dv = plsc.load_gather(rd, [eidx, dd])
                return acc + sv * dv

            acc = lax.fori_loop(0, D, dbody, jnp.zeros((16,), jnp.float32),
                                unroll=8)
            res = 1.0 / (1.0 + jnp.exp(-acc))
            out_v[pl.ds(g * C + grp * 16, 16)] = res

    for b in range(NBUF):
        start(b, b)

    def ring(q, carry):
        for b in range(NBUF):
            g = NBUF * q + b
            compute(g, b)

            @pl.when(g + NBUF < NCH)
            def _():
                start(g + NBUF, b)

        return carry

    lax.fori_loop(0, NCH // NBUF, ring, 0)
    for g in range((NCH // NBUF) * NBUF, NCH):
        compute(g, g % NBUF)

    pltpu.sync_copy(out_v, out.at[pl.ds(wid * E_PER_W, E_PER_W)])


@functools.partial(jax.jit, donate_argnums=())
def _decode(table, src_i, dst_i):
    run = functools.partial(
        pl.kernel,
        out_type=jax.ShapeDtypeStruct((B,), jnp.float32),
        mesh=plsc.VectorSubcoreMesh(core_axis_name="c", subcore_axis_name="s"),
        compiler_params=pltpu.CompilerParams(
            needs_layout_passes=False, use_tc_tiling_on_sc=False),
        scratch_types=[
            pltpu.VMEM((NCH, C), jnp.int32),
            pltpu.VMEM((NCH, C), jnp.int32),
            pltpu.VMEM((NBUF, C, D), jnp.float32),
            pltpu.VMEM((NBUF, C, D), jnp.float32),
            pltpu.VMEM((E_PER_W,), jnp.float32),
        ] + [pltpu.SemaphoreType.DMA] * (2 * NBUF),
    )(_body)
    return run(table, src_i, dst_i)


def kernel(quantized_latent_embedding, edge_index):
    src_i = edge_index[0].astype(jnp.int32).reshape(NW, NCH, C)
    dst_i = edge_index[1].astype(jnp.int32).reshape(NW, NCH, C)
    return _decode(quantized_latent_embedding, src_i, dst_i)


# speedup vs baseline: 10.9347x; 1.0056x over previous
import functools

import jax
import jax.numpy as jnp
from jax import lax
from jax.experimental import pallas as pl
from jax.experimental.pallas import tpu as pltpu
from jax.experimental.pallas import tpu_sc as plsc

V = 10000
D = 128
B = 320000
NC, NS = 2, 16
NW = NC * NS
E_PER_W = B // NW
C = 80
NCH = E_PER_W // C
GRP = C // 16


NBUF = 4


def _body(table, src_i, dst_i, out, idx_s, idx_d, rows_s, rows_d, out_v,
          *sems_flat):
    wid = lax.axis_index("c") * NS + lax.axis_index("s")

    pltpu.sync_copy(src_i.at[wid], idx_s)
    pltpu.sync_copy(dst_i.at[wid], idx_d)

    sems = tuple(zip(sems_flat[:NBUF], sems_flat[NBUF:]))

    def start(g, b):
        ss, sd = sems[b]
        pltpu.async_copy(table.at[idx_s.at[g]], rows_s.at[b], ss)
        pltpu.async_copy(table.at[idx_d.at[g]], rows_d.at[b], sd)

    def compute(g, b):
        ss, sd = sems[b]
        pltpu.make_async_copy(table.at[idx_s.at[g]], rows_s.at[b], ss).wait()
        pltpu.make_async_copy(table.at[idx_d.at[g]], rows_d.at[b], sd).wait()
        rs = rows_s.at[b]
        rd = rows_d.at[b]
        lanes = jnp.arange(16, dtype=jnp.int32)
        for grp in range(GRP):
            eidx = lanes + (grp * 16)

            def dbody(i, accs):
                out = []
                for j, acc in enumerate(accs):
                    d = i * 8 + j
                    dd = (lanes + d) & 127
                    sv = plsc.load_gather(rs, [eidx, dd])
                    dv = plsc.load_gather(rd, [eidx, dd])
                    out.append(acc + sv * dv)
                return tuple(out)

            zero = jnp.zeros((16,), jnp.float32)
            accs = lax.fori_loop(0, D // 8, dbody, (zero,) * 8)
            a = ((accs[0] + accs[1]) + (accs[2] + accs[3])) + (
                (accs[4] + accs[5]) + (accs[6] + accs[7]))
            res = 1.0 / (1.0 + jnp.exp(-a))
            out_v[pl.ds(g * C + grp * 16, 16)] = res

    for b in range(NBUF):
        start(b, b)

    def ring(q, carry):
        for b in range(NBUF):
            g = NBUF * q + b
            compute(g, b)

            @pl.when(g + NBUF < NCH)
            def _():
                start(g + NBUF, b)

        return carry

    lax.fori_loop(0, NCH // NBUF, ring, 0)
    for g in range((NCH // NBUF) * NBUF, NCH):
        compute(g, g % NBUF)

    pltpu.sync_copy(out_v, out.at[pl.ds(wid * E_PER_W, E_PER_W)])


@functools.partial(jax.jit, donate_argnums=())
def _decode(table, src_i, dst_i):
    run = functools.partial(
        pl.kernel,
        out_type=jax.ShapeDtypeStruct((B,), jnp.float32),
        mesh=plsc.VectorSubcoreMesh(core_axis_name="c", subcore_axis_name="s"),
        compiler_params=pltpu.CompilerParams(
            needs_layout_passes=False, use_tc_tiling_on_sc=False),
        scratch_types=[
            pltpu.VMEM((NCH, C), jnp.int32),
            pltpu.VMEM((NCH, C), jnp.int32),
            pltpu.VMEM((NBUF, C, D), jnp.float32),
            pltpu.VMEM((NBUF, C, D), jnp.float32),
            pltpu.VMEM((E_PER_W,), jnp.float32),
        ] + [pltpu.SemaphoreType.DMA] * (2 * NBUF),
    )(_body)
    return run(table, src_i, dst_i)


def kernel(quantized_latent_embedding, edge_index):
    src_i = edge_index[0].astype(jnp.int32).reshape(NW, NCH, C)
    dst_i = edge_index[1].astype(jnp.int32).reshape(NW, NCH, C)
    return _decode(quantized_latent_embedding, src_i, dst_i)
